# baseline (speedup 1.0000x reference)
import functools

import jax
import jax.numpy as jnp
from jax import lax
from jax.experimental import pallas as pl
from jax.experimental.pallas import tpu as pltpu
from jax.experimental.pallas import tpu_sc as plsc

NC = 2
NS = 16
NW = NC * NS
L = 16
IDX_CHUNK = 128


@functools.lru_cache(maxsize=None)
def _make_gather_kernel(B, D):
    b_per_w = B // NW
    n_grp = b_per_w // L
    n_chunk = b_per_w // IDX_CHUNK
    mesh = plsc.VectorSubcoreMesh(core_axis_name="c", subcore_axis_name="s")

    @functools.partial(
        pl.kernel,
        mesh=mesh,
        out_type=jax.ShapeDtypeStruct((B * D,), jnp.float32),
        compiler_params=pltpu.CompilerParams(needs_layout_passes=False),
        scratch_types=[
            pltpu.VMEM((n_chunk, IDX_CHUNK), jnp.int32),
            pltpu.VMEM((b_per_w,), jnp.int32),
            pltpu.VMEM((b_per_w, 2 * D), jnp.float32),
            pltpu.VMEM((b_per_w * D,), jnp.float32),
            pltpu.SemaphoreType.DMA,
        ],
    )
    def gather_kernel(ids_hbm, w2_hbm, out_hbm, idx, off, rows2, rows, s0):
        wid = lax.axis_index("s") * NC + lax.axis_index("c")
        base = wid * b_per_w

        for t in range(n_chunk):
            pltpu.sync_copy(ids_hbm.at[pl.ds(base + t * IDX_CHUNK, IDX_CHUNK)],
                            idx.at[t])

        spc = IDX_CHUNK // L

        def sub_one(j, carry):
            t = j // spc
            o = (j % spc) * L
            v = idx[t, pl.ds(o, L)] - 1
            off[pl.ds(j * L, L)] = jnp.bitwise_and(v, 1) * D
            idx[t, pl.ds(o, L)] = jnp.right_shift(v, 1)
            return carry
        lax.fori_loop(0, n_grp, sub_one, 0)

        copies = []
        for t in range(n_chunk):
            r = pl.ds(t * IDX_CHUNK, IDX_CHUNK)
            copies.append(pltpu.async_copy(w2_hbm.at[idx.at[t]], rows2.at[r], s0))
        for cp in copies:
            cp.wait()

        def extract(j, carry):
            ov = off[pl.ds(j * L, L)]
            for k in range(L):
                row = j * L + k
                o = ov[k]
                for c in range(D // L):
                    rows[pl.ds(row * D + c * L, L)] = \
                        rows2[row, pl.ds(o + c * L, L)]
            return carry
        lax.fori_loop(0, n_grp, extract, 0)

        pltpu.sync_copy(rows, out_hbm.at[pl.ds(base * D, b_per_w * D)])

    return gather_kernel


@functools.lru_cache(maxsize=None)
def _make_combine_kernel(B, D):
    b_per_w = B // NW
    n_grp = b_per_w // L
    mesh = plsc.VectorSubcoreMesh(core_axis_name="c", subcore_axis_name="s")

    @functools.partial(
        pl.kernel,
        mesh=mesh,
        out_type=jax.ShapeDtypeStruct((B,), jnp.float32),
        compiler_params=pltpu.CompilerParams(
            use_tc_tiling_on_sc=False, needs_layout_passes=False),
        scratch_types=[
            pltpu.VMEM((b_per_w, D), jnp.float32),
            pltpu.VMEM((b_per_w, D), jnp.float32),
            pltpu.VMEM((b_per_w,), jnp.float32),
            pltpu.VMEM((b_per_w,), jnp.float32),
            pltpu.VMEM((b_per_w,), jnp.float32),
            pltpu.SemaphoreType.DMA,
            pltpu.SemaphoreType.DMA,
        ],
    )
    def combine_kernel(ue_hbm, ie_hbm, ub_hbm, ib_hbm,
                       out_hbm, urows, irows, ubv, ibv, outv, s0, s1):
        wid = lax.axis_index("s") * NC + lax.axis_index("c")
        base = wid * b_per_w

        cps = [
            pltpu.async_copy(ue_hbm.at[pl.ds(base, b_per_w), :], urows, s0),
            pltpu.async_copy(ie_hbm.at[pl.ds(base, b_per_w), :], irows, s1),
        ]
        pltpu.sync_copy(ub_hbm.at[pl.ds(base, b_per_w)], ubv)
        pltpu.sync_copy(ib_hbm.at[pl.ds(base, b_per_w)], ibv)
        for cp in cps:
            cp.wait()

        lane = lax.iota(jnp.int32, L)
        mask_lo = lane < (L // 2)
        half = lane & (L // 2 - 1)
        fold_idx = {w: lane ^ w for w in (8, 4, 2, 1)}
        pack_idx = {w: (half // w) * (2 * w) + (half % w) for w in (8, 4, 2, 1)}

        gdn = lax.GatherDimensionNumbers(
            offset_dims=(), collapsed_slice_dims=(0,), start_index_map=(0,))

        def take(v, idx):
            return lax.gather(v, idx[:, None], dimension_numbers=gdn,
                              slice_sizes=(1,), unique_indices=True,
                              indices_are_sorted=False,
                              mode=lax.GatherScatterMode.PROMISE_IN_BOUNDS)

        def fold_pair(a, b, w):
            fa = a + take(a, fold_idx[w])
            fb = b + take(b, fold_idx[w])
            return jnp.where(mask_lo, take(fa, pack_idx[w]),
                             take(fb, pack_idx[w]))

        def group(g, carry):
            svecs = []
            for b in range(L):
                row = g * L + b
                acc = urows[row, pl.ds(0, L)] * irows[row, pl.ds(0, L)]
                for c in range(1, D // L):
                    acc = acc + (urows[row, pl.ds(c * L, L)]
                                 * irows[row, pl.ds(c * L, L)])
                svecs.append(acc)
            w = L // 2
            while len(svecs) > 1:
                svecs = [fold_pair(svecs[2 * i], svecs[2 * i + 1], w)
                         for i in range(len(svecs) // 2)]
                w //= 2
            res = svecs[0] + ubv[pl.ds(g * L, L)] + ibv[pl.ds(g * L, L)]
            outv[pl.ds(g * L, L)] = 5.0 / (1.0 + jnp.exp(-res))
            return carry
        lax.fori_loop(0, n_grp, group, 0)

        pltpu.sync_copy(outv, out_hbm.at[pl.ds(base, b_per_w)])

    return combine_kernel


def kernel(users, items, u_weight, i_weight, u_bias, i_bias):
    B = users.shape[0]
    D = u_weight.shape[1]
    gk = _make_gather_kernel(B, D)
    u_emb = gk(users, u_weight.reshape(-1, 2 * D)).reshape(B, D)
    i_emb = gk(items, i_weight.reshape(-1, 2 * D)).reshape(B, D)
    ub_g = jnp.take(u_bias, users - 1, axis=0).reshape(-1)
    ib_g = jnp.take(i_bias, items - 1, axis=0).reshape(-1)
    return _make_combine_kernel(B, D)(u_emb, i_emb, ub_g, ib_g)


# speedup vs baseline: 1.0150x; 1.0150x over previous
import functools

import jax
import jax.numpy as jnp
from jax import lax
from jax.experimental import pallas as pl
from jax.experimental.pallas import tpu as pltpu
from jax.experimental.pallas import tpu_sc as plsc

NC = 2
NS = 16
NW = NC * NS
L = 16
IDX_CHUNK = 128


@functools.lru_cache(maxsize=None)
def _make_kernel(B, D):
    b_per_w = B // NW
    n_grp = b_per_w // L
    n_chunk = b_per_w // IDX_CHUNK
    mesh = plsc.VectorSubcoreMesh(core_axis_name="c", subcore_axis_name="s")

    @functools.partial(
        pl.kernel,
        mesh=mesh,
        out_type=jax.ShapeDtypeStruct((B,), jnp.float32),
        compiler_params=pltpu.CompilerParams(
            use_tc_tiling_on_sc=False, needs_layout_passes=False),
        scratch_types=[
            pltpu.VMEM((n_chunk, IDX_CHUNK), jnp.int32),
            pltpu.VMEM((n_chunk, IDX_CHUNK), jnp.int32),
            pltpu.VMEM((b_per_w, D), jnp.float32),
            pltpu.VMEM((b_per_w, D), jnp.float32),
            pltpu.VMEM((b_per_w,), jnp.float32),
            pltpu.VMEM((b_per_w,), jnp.float32),
            pltpu.VMEM((b_per_w,), jnp.float32),
            pltpu.SemaphoreType.DMA,
            pltpu.SemaphoreType.DMA,
            pltpu.SemaphoreType.DMA,
            pltpu.SemaphoreType.DMA,
        ],
    )
    def net_kernel(users_hbm, items_hbm, uw_hbm, iw_hbm, ub_hbm, ib_hbm,
                   out_hbm, uidx, iidx, urows, irows, ubv, ibv, outv,
                   s0, s1, s2, s3):
        wid = lax.axis_index("s") * NC + lax.axis_index("c")
        base = wid * b_per_w

        for t in range(n_chunk):
            pltpu.sync_copy(users_hbm.at[pl.ds(base + t * IDX_CHUNK, IDX_CHUNK)],
                            uidx.at[t])
            pltpu.sync_copy(items_hbm.at[pl.ds(base + t * IDX_CHUNK, IDX_CHUNK)],
                            iidx.at[t])

        spc = IDX_CHUNK // L

        def sub_one(j, carry):
            t = j // spc
            o = (j % spc) * L
            uidx[t, pl.ds(o, L)] = uidx[t, pl.ds(o, L)] - 1
            iidx[t, pl.ds(o, L)] = iidx[t, pl.ds(o, L)] - 1
            return carry
        lax.fori_loop(0, n_grp, sub_one, 0)

        copies = []
        for t in range(n_chunk):
            r = pl.ds(t * IDX_CHUNK, IDX_CHUNK)
            copies.append(pltpu.async_copy(uw_hbm.at[uidx.at[t]], urows.at[r], s0))
            copies.append(pltpu.async_copy(iw_hbm.at[iidx.at[t]], irows.at[r], s1))
            copies.append(pltpu.async_copy(ub_hbm.at[uidx.at[t]], ubv.at[r], s2))
            copies.append(pltpu.async_copy(ib_hbm.at[iidx.at[t]], ibv.at[r], s3))
        for cp in copies:
            cp.wait()

        lane = lax.iota(jnp.int32, L)
        mask_lo = lane < (L // 2)
        half = lane & (L // 2 - 1)
        fold_idx = {w: lane ^ w for w in (8, 4, 2, 1)}
        pack_idx = {w: (half // w) * (2 * w) + (half % w) for w in (8, 4, 2, 1)}

        gdn = lax.GatherDimensionNumbers(
            offset_dims=(), collapsed_slice_dims=(0,), start_index_map=(0,))

        def take(v, idx):
            return lax.gather(v, idx[:, None], dimension_numbers=gdn,
                              slice_sizes=(1,), unique_indices=True,
                              indices_are_sorted=False,
                              mode=lax.GatherScatterMode.PROMISE_IN_BOUNDS)

        def fold_pair(a, b, w):
            fa = a + take(a, fold_idx[w])
            fb = b + take(b, fold_idx[w])
            return jnp.where(mask_lo, take(fa, pack_idx[w]),
                             take(fb, pack_idx[w]))

        def group(g, carry):
            svecs = []
            for b in range(L):
                row = g * L + b
                acc = urows[row, pl.ds(0, L)] * irows[row, pl.ds(0, L)]
                for c in range(1, D // L):
                    acc = acc + (urows[row, pl.ds(c * L, L)]
                                 * irows[row, pl.ds(c * L, L)])
                svecs.append(acc)
            w = L // 2
            while len(svecs) > 1:
                svecs = [fold_pair(svecs[2 * i], svecs[2 * i + 1], w)
                         for i in range(len(svecs) // 2)]
                w //= 2
            res = svecs[0] + ubv[pl.ds(g * L, L)] + ibv[pl.ds(g * L, L)]
            outv[pl.ds(g * L, L)] = 5.0 / (1.0 + jnp.exp(-res))
            return carry
        lax.fori_loop(0, n_grp, group, 0)

        pltpu.sync_copy(outv, out_hbm.at[pl.ds(base, b_per_w)])

    return net_kernel


def kernel(users, items, u_weight, i_weight, u_bias, i_bias):
    B = users.shape[0]
    D = u_weight.shape[1]
    k = _make_kernel(B, D)
    return k(users, items, u_weight, i_weight,
             u_bias.reshape(-1), i_bias.reshape(-1))
